# baseline (device time: 32105 ns/iter reference)
import jax
import jax.numpy as jnp
from jax import lax
from jax.experimental import pallas as pl
from jax.experimental.pallas import tpu as pltpu

N_DEV = 4
SQ = 256
DH = 128
HQ = 8
HKV = 2
G = HQ // HKV
SCALE = 0.08838834764831843
CROWS = DH + 8

_MINE, _FL1, _FR1, _FL2 = 0, 1, 2, 3


def kernel(x, Wq, Wo, K_ext, V_ext):
    skv = K_ext.shape[1]
    x2 = x.reshape(SQ, HQ * DH)
    K2 = K_ext.reshape(skv, HKV * DH)
    V2 = V_ext.reshape(skv, HKV * DH)

    def body(x_ref, wq_ref, wo_ref, k_ref, v_ref, out_ref,
             comm_ref, send_sems, recv_sems):
        my = lax.axis_index("i")
        left = lax.rem(my + N_DEV - 1, N_DEV)
        right = lax.rem(my + 1, N_DEV)

        barrier_sem = pltpu.get_barrier_semaphore()
        for nbr in (left, right):
            pl.semaphore_signal(
                barrier_sem, inc=1,
                device_id=(nbr,), device_id_type=pl.DeviceIdType.MESH,
            )
        pl.semaphore_wait(barrier_sem, 2)

        def desc(src_slot, dst_slot, sem, target):
            return pltpu.make_async_remote_copy(
                src_ref=comm_ref.at[src_slot],
                dst_ref=comm_ref.at[dst_slot],
                send_sem=send_sems.at[sem],
                recv_sem=recv_sems.at[sem],
                device_id=(target,),
                device_id_type=pl.DeviceIdType.MESH,
            )

        dA = [desc(_MINE * HQ + h, _FL1 * HQ + h, h, right) for h in range(HQ)]
        dB = [desc(_MINE * HQ + h, _FR1 * HQ + h, HQ + h, left) for h in range(HQ)]
        dC = [desc(_FL1 * HQ + h, _FL2 * HQ + h, 2 * HQ + h, right) for h in range(HQ)]

        xb = x_ref[...].astype(jnp.bfloat16)
        wqb = wq_ref[...].astype(jnp.bfloat16)
        q = jnp.dot(xb, wqb, preferred_element_type=jnp.float32) * SCALE
        qb = q.astype(jnp.bfloat16)

        kb = [k_ref[:, g * DH:(g + 1) * DH].astype(jnp.bfloat16)
              for g in range(HKV)]
        vb = [v_ref[:, g * DH:(g + 1) * DH].astype(jnp.bfloat16)
              for g in range(HKV)]
        for h in range(HQ):
            g = h // G
            qh = qb[:, h * DH:(h + 1) * DH]
            st = lax.dot_general(kb[g], qh, (((1,), (1,)), ((), ())),
                                 preferred_element_type=jnp.float32)
            pt = jnp.exp(st)
            lsum = jnp.sum(pt, axis=0, keepdims=True)
            ogt = lax.dot_general(vb[g], pt.astype(jnp.bfloat16),
                                  (((0,), (0,)), ((), ())),
                                  preferred_element_type=jnp.float32)
            comm_ref[_MINE * HQ + h, 0:DH, :] = ogt.astype(jnp.bfloat16)
            comm_ref[_MINE * HQ + h, DH:DH + 1, :] = lsum.astype(jnp.bfloat16)
            dA[h].start()
            dB[h].start()

        for h in range(HQ):
            dA[h].wait_recv()
            dC[h].start()

        y = None
        for h in range(HQ):
            dB[h].wait_recv()
            dC[h].wait_recv()
            acc = ((comm_ref[_MINE * HQ + h].astype(jnp.float32)
                    + comm_ref[_FL1 * HQ + h].astype(jnp.float32))
                   + (comm_ref[_FR1 * HQ + h].astype(jnp.float32)
                      + comm_ref[_FL2 * HQ + h].astype(jnp.float32)))
            outn = (acc[0:DH, :] / acc[DH:DH + 1, :]).astype(jnp.bfloat16)
            attn_h = jnp.transpose(outn)
            yh = jnp.dot(attn_h,
                         wo_ref[h * DH:(h + 1) * DH, :].astype(jnp.bfloat16),
                         preferred_element_type=jnp.float32)
            y = yh if y is None else y + yh
        out_ref[...] = y

        for d in dA + dB + dC:
            d.wait_send()

    out = pl.pallas_call(
        body,
        out_shape=jax.ShapeDtypeStruct((SQ, HQ * DH), jnp.float32),
        in_specs=[pl.BlockSpec(memory_space=pltpu.VMEM)] * 5,
        out_specs=pl.BlockSpec(memory_space=pltpu.VMEM),
        scratch_shapes=[
            pltpu.VMEM((4 * HQ, CROWS, SQ), jnp.bfloat16),
            pltpu.SemaphoreType.DMA((3 * HQ,)),
            pltpu.SemaphoreType.DMA((3 * HQ,)),
        ],
        compiler_params=pltpu.CompilerParams(collective_id=0),
    )(x2, Wq, Wo, K2, V2)
    return out.reshape(1, SQ, HQ * DH)


# device time: 29305 ns/iter; 1.0955x vs baseline; 1.0955x over previous
import jax
import jax.numpy as jnp
from jax import lax
from jax.experimental import pallas as pl
from jax.experimental.pallas import tpu as pltpu

N_DEV = 4
SQ = 256
DH = 128
HQ = 8
HKV = 2
G = HQ // HKV
SCALE = 0.08838834764831843
NC = 4
HPC = HQ // NC
CCOLS = HPC * SQ
CROWS = DH + 8

_MINE, _FL1, _FR1, _FL2 = 0, 1, 2, 3


def kernel(x, Wq, Wo, K_ext, V_ext):
    skv = K_ext.shape[1]
    x2 = x.reshape(SQ, HQ * DH)
    K2 = K_ext.reshape(skv, HKV * DH)
    V2 = V_ext.reshape(skv, HKV * DH)

    def body(x_ref, wq_ref, wo_ref, k_ref, v_ref, out_ref,
             comm_ref, send_sems, recv_sems):
        my = lax.axis_index("i")
        left = lax.rem(my + N_DEV - 1, N_DEV)
        right = lax.rem(my + 1, N_DEV)

        barrier_sem = pltpu.get_barrier_semaphore()
        for nbr in (left, right):
            pl.semaphore_signal(
                barrier_sem, inc=1,
                device_id=(nbr,), device_id_type=pl.DeviceIdType.MESH,
            )
        pl.semaphore_wait(barrier_sem, 2)

        def desc(src_slot, dst_slot, sem, target):
            return pltpu.make_async_remote_copy(
                src_ref=comm_ref.at[src_slot],
                dst_ref=comm_ref.at[dst_slot],
                send_sem=send_sems.at[sem],
                recv_sem=recv_sems.at[sem],
                device_id=(target,),
                device_id_type=pl.DeviceIdType.MESH,
            )

        dA = [desc(_MINE * NC + c, _FL1 * NC + c, c, right) for c in range(NC)]
        dB = [desc(_MINE * NC + c, _FR1 * NC + c, NC + c, left) for c in range(NC)]
        dC = [desc(_FL1 * NC + c, _FL2 * NC + c, 2 * NC + c, right) for c in range(NC)]

        xb = x_ref[...].astype(jnp.bfloat16)
        wqb = wq_ref[...].astype(jnp.bfloat16)
        q = jnp.dot(xb, wqb, preferred_element_type=jnp.float32) * SCALE
        qb = q.astype(jnp.bfloat16)

        kb = [k_ref[:, g * DH:(g + 1) * DH].astype(jnp.bfloat16)
              for g in range(HKV)]
        vb = [v_ref[:, g * DH:(g + 1) * DH].astype(jnp.bfloat16)
              for g in range(HKV)]
        for c in range(NC):
            g = (c * HPC) // G
            qc = jnp.concatenate(
                [qb[:, (c * HPC + j) * DH:(c * HPC + j + 1) * DH]
                 for j in range(HPC)],
                axis=0,
            )
            st = lax.dot_general(kb[g], qc, (((1,), (1,)), ((), ())),
                                 preferred_element_type=jnp.float32)
            pt = jnp.exp(st)
            lsum = jnp.sum(pt, axis=0, keepdims=True)
            ogt = lax.dot_general(vb[g], pt.astype(jnp.bfloat16),
                                  (((0,), (0,)), ((), ())),
                                  preferred_element_type=jnp.float32)
            comm_ref[_MINE * NC + c, 0:DH, :] = ogt.astype(jnp.bfloat16)
            comm_ref[_MINE * NC + c, DH:DH + 1, :] = lsum.astype(jnp.bfloat16)
            dA[c].start()
            dB[c].start()

        for c in range(NC):
            dA[c].wait_recv()
            dC[c].start()

        y = None
        for c in range(NC):
            dB[c].wait_recv()
            dC[c].wait_recv()
            acc = ((comm_ref[_MINE * NC + c].astype(jnp.float32)
                    + comm_ref[_FL1 * NC + c].astype(jnp.float32))
                   + (comm_ref[_FR1 * NC + c].astype(jnp.float32)
                      + comm_ref[_FL2 * NC + c].astype(jnp.float32)))
            outn = (acc[0:DH, :] / acc[DH:DH + 1, :]).astype(jnp.bfloat16)
            attn_c = jnp.concatenate(
                [jnp.transpose(outn[:, j * SQ:(j + 1) * SQ])
                 for j in range(HPC)],
                axis=1,
            )
            yc = jnp.dot(
                attn_c,
                wo_ref[c * HPC * DH:(c + 1) * HPC * DH, :].astype(jnp.bfloat16),
                preferred_element_type=jnp.float32)
            y = yc if y is None else y + yc
        out_ref[...] = y

        for d in dA + dB + dC:
            d.wait_send()

    out = pl.pallas_call(
        body,
        out_shape=jax.ShapeDtypeStruct((SQ, HQ * DH), jnp.float32),
        in_specs=[pl.BlockSpec(memory_space=pltpu.VMEM)] * 5,
        out_specs=pl.BlockSpec(memory_space=pltpu.VMEM),
        scratch_shapes=[
            pltpu.VMEM((4 * NC, CROWS, CCOLS), jnp.bfloat16),
            pltpu.SemaphoreType.DMA((3 * NC,)),
            pltpu.SemaphoreType.DMA((3 * NC,)),
        ],
        compiler_params=pltpu.CompilerParams(collective_id=0),
    )(x2, Wq, Wo, K2, V2)
    return out.reshape(1, SQ, HQ * DH)


# device time: 28688 ns/iter; 1.1191x vs baseline; 1.0215x over previous
import jax
import jax.numpy as jnp
from jax import lax
from jax.experimental import pallas as pl
from jax.experimental.pallas import tpu as pltpu

N_DEV = 4
SQ = 256
DH = 128
HQ = 8
HKV = 2
G = HQ // HKV
SCALE = 0.08838834764831843
NC = 4
HPC = HQ // NC
CCOLS = HPC * SQ
CROWS = DH + 8

_MINE, _P1, _SUM, _P2 = 0, 1, 2, 3


def kernel(x, Wq, Wo, K_ext, V_ext):
    skv = K_ext.shape[1]
    x2 = x.reshape(SQ, HQ * DH)
    K2 = K_ext.reshape(skv, HKV * DH)
    V2 = V_ext.reshape(skv, HKV * DH)

    def body(x_ref, wq_ref, wo_ref, k_ref, v_ref, out_ref,
             comm_ref, send_sems, recv_sems):
        my = lax.axis_index("i")
        p1 = my + 1 - 2 * lax.rem(my, 2)
        p2 = (N_DEV - 1) - my

        barrier_sem = pltpu.get_barrier_semaphore()
        for nbr in (p1, p2):
            pl.semaphore_signal(
                barrier_sem, inc=1,
                device_id=(nbr,), device_id_type=pl.DeviceIdType.MESH,
            )
        pl.semaphore_wait(barrier_sem, 2)

        def desc(src_slot, dst_slot, sem, target):
            return pltpu.make_async_remote_copy(
                src_ref=comm_ref.at[src_slot],
                dst_ref=comm_ref.at[dst_slot],
                send_sem=send_sems.at[sem],
                recv_sem=recv_sems.at[sem],
                device_id=(target,),
                device_id_type=pl.DeviceIdType.MESH,
            )

        d1 = [desc(_MINE * NC + c, _P1 * NC + c, c, p1) for c in range(NC)]
        d2 = [desc(_SUM * NC + c, _P2 * NC + c, NC + c, p2) for c in range(NC)]

        def fwd(c):
            d1[c].wait_recv()
            pair = (comm_ref[_MINE * NC + c].astype(jnp.float32)
                    + comm_ref[_P1 * NC + c].astype(jnp.float32))
            comm_ref[_SUM * NC + c, :, :] = pair.astype(jnp.bfloat16)
            d2[c].start()

        xb = x_ref[...].astype(jnp.bfloat16)
        wqb = wq_ref[...].astype(jnp.bfloat16)
        q = jnp.dot(xb, wqb, preferred_element_type=jnp.float32) * SCALE
        qb = q.astype(jnp.bfloat16)

        kb = [k_ref[:, g * DH:(g + 1) * DH].astype(jnp.bfloat16)
              for g in range(HKV)]
        vb = [v_ref[:, g * DH:(g + 1) * DH].astype(jnp.bfloat16)
              for g in range(HKV)]
        for c in range(NC):
            g = (c * HPC) // G
            qc = jnp.concatenate(
                [qb[:, (c * HPC + j) * DH:(c * HPC + j + 1) * DH]
                 for j in range(HPC)],
                axis=0,
            )
            st = lax.dot_general(kb[g], qc, (((1,), (1,)), ((), ())),
                                 preferred_element_type=jnp.float32)
            pt = jnp.exp(st)
            lsum = jnp.sum(pt, axis=0, keepdims=True)
            ogt = lax.dot_general(vb[g], pt.astype(jnp.bfloat16),
                                  (((0,), (0,)), ((), ())),
                                  preferred_element_type=jnp.float32)
            comm_ref[_MINE * NC + c, 0:DH, :] = ogt.astype(jnp.bfloat16)
            comm_ref[_MINE * NC + c, DH:DH + 1, :] = lsum.astype(jnp.bfloat16)
            d1[c].start()
            if c >= 1:
                fwd(c - 1)
        fwd(NC - 1)

        y = None
        for c in range(NC):
            d2[c].wait_recv()
            acc = (comm_ref[_SUM * NC + c].astype(jnp.float32)
                   + comm_ref[_P2 * NC + c].astype(jnp.float32))
            outn = (acc[0:DH, :] / acc[DH:DH + 1, :]).astype(jnp.bfloat16)
            attn_c = jnp.concatenate(
                [jnp.transpose(outn[:, j * SQ:(j + 1) * SQ])
                 for j in range(HPC)],
                axis=1,
            )
            yc = jnp.dot(
                attn_c,
                wo_ref[c * HPC * DH:(c + 1) * HPC * DH, :].astype(jnp.bfloat16),
                preferred_element_type=jnp.float32)
            y = yc if y is None else y + yc
        out_ref[...] = y

        for d in d1 + d2:
            d.wait_send()

    out = pl.pallas_call(
        body,
        out_shape=jax.ShapeDtypeStruct((SQ, HQ * DH), jnp.float32),
        in_specs=[pl.BlockSpec(memory_space=pltpu.VMEM)] * 5,
        out_specs=pl.BlockSpec(memory_space=pltpu.VMEM),
        scratch_shapes=[
            pltpu.VMEM((4 * NC, CROWS, CCOLS), jnp.bfloat16),
            pltpu.SemaphoreType.DMA((2 * NC,)),
            pltpu.SemaphoreType.DMA((2 * NC,)),
        ],
        compiler_params=pltpu.CompilerParams(collective_id=0),
    )(x2, Wq, Wo, K2, V2)
    return out.reshape(1, SQ, HQ * DH)
